# baseline (speedup 1.0000x reference)
import functools

import jax
import jax.numpy as jnp
from jax import lax
from jax.experimental import pallas as pl
from jax.experimental.pallas import tpu as pltpu
from jax.experimental.pallas import tpu_sc as plsc

BITS = 24
B = 16384
D = 512
LANES = 128
BLK = 2048

_NC = 2
_NS = 16
_L = 16
_NW = _NC * _NS
_PER = B // _NW
_CHUNK = 128
_NSTREAM = _PER // _CHUNK


def _hash_tc(x_ref, rm_ref, pw_ref, idx_ref, ridx_ref):
    prod = jnp.dot(x_ref[...], rm_ref[...],
                   preferred_element_type=jnp.float32)
    vals = jnp.where(prod < 0.0, pw_ref[...], 0)
    idx = jnp.sum(vals, axis=1, keepdims=True)
    idx_ref[...] = idx
    ridx_ref[...] = lax.shift_right_logical(idx, 12)


def _tc_stage(x, rm_padded, pw):
    grid = (B // BLK,)
    idx, ridx = pl.pallas_call(
        _hash_tc,
        grid=grid,
        in_specs=[
            pl.BlockSpec((BLK, D), lambda i: (i, 0)),
            pl.BlockSpec((D, LANES), lambda i: (0, 0)),
            pl.BlockSpec((1, LANES), lambda i: (0, 0)),
        ],
        out_specs=[
            pl.BlockSpec((BLK, 1), lambda i: (i, 0)),
            pl.BlockSpec((BLK, 1), lambda i: (i, 0)),
        ],
        out_shape=[
            jax.ShapeDtypeStruct((B, 1), jnp.int32),
            jax.ShapeDtypeStruct((B, 1), jnp.int32),
        ],
    )(x, rm_padded, pw)
    return idx.reshape(B), ridx.reshape(B)


_ROW = 128


def _sc_gather_body(bs_hbm, idx_hbm, ridx_hbm, out_hbm,
                    idx_v, ridx_v, rows_v, out_v, sem):
    table = bs_hbm.bitcast(jnp.int32)
    wid = lax.axis_index("s") * _NC + lax.axis_index("c")
    base = wid * _PER
    lane = lax.iota(jnp.int32, _L)
    for j in range(_NSTREAM):
        pltpu.sync_copy(idx_hbm.at[pl.ds(base + j * _CHUNK, _CHUNK)],
                        idx_v.at[j])
        pltpu.sync_copy(ridx_hbm.at[pl.ds(base + j * _CHUNK, _CHUNK)],
                        ridx_v.at[j])
    cps = [pltpu.async_copy(table.at[ridx_v.at[j]], rows_v.at[j], sem)
           for j in range(_NSTREAM)]
    for cp in cps:
        cp.wait()

    def _group(jt, carry):
        j = lax.shift_right_logical(jt, 3)
        t = jt & 7
        iv = idx_v[j, pl.ds(t * _L, _L)]
        cv = lax.shift_right_logical(iv, 3) & (_ROW - 1)
        gv = lax.shift_right_logical(cv, 4)
        pv = cv & (_L - 1)
        acc = jnp.zeros((_L,), jnp.int32)
        for r in range(_L):
            g_r = gv[r]
            sub = jnp.zeros((_L,), jnp.int32)
            for g in range(_ROW // _L):
                v = rows_v[j, t * _L + r, pl.ds(g * _L, _L)]
                sub = jnp.where(g_r == g, v, sub)
            acc = jnp.where(lane == r, sub[pv], acc)
        bitpos = lax.shift_left(lax.shift_right_logical(iv, 10) & 3, 3) \
            | (iv & 7)
        m = lax.shift_left(jnp.full((_L,), 1, jnp.int32), bitpos)
        out_v[j, pl.ds(t * _L, _L)] = jnp.where((acc & m) != 0, 1, 0)
        return carry

    lax.fori_loop(0, _NSTREAM * (_CHUNK // _L), _group, 0)
    for j in range(_NSTREAM):
        pltpu.sync_copy(out_v.at[j],
                        out_hbm.at[pl.ds(base + j * _CHUNK, _CHUNK)])


@functools.lru_cache(maxsize=None)
def _make_sc_gather():
    return pl.kernel(
        _sc_gather_body,
        mesh=plsc.VectorSubcoreMesh(core_axis_name="c", subcore_axis_name="s"),
        out_type=jax.ShapeDtypeStruct((B,), jnp.int32),
        scratch_types=[
            pltpu.VMEM((_NSTREAM, _CHUNK), jnp.int32),
            pltpu.VMEM((_NSTREAM, _CHUNK), jnp.int32),
            pltpu.VMEM((_NSTREAM, _CHUNK, _ROW), jnp.int32),
            pltpu.VMEM((_NSTREAM, _CHUNK), jnp.int32),
            pltpu.SemaphoreType.DMA,
        ],
    )


def kernel(x, random_matrix, binary_set, is_training, test_local_stats):
    x2 = jnp.reshape(x, (B, D))
    rm_padded = jnp.pad(random_matrix, ((0, 0), (0, LANES - BITS)))
    pw = jnp.pad((2 ** jnp.arange(BITS, dtype=jnp.int32))[None, :],
                 ((0, 0), (0, LANES - BITS)))
    idx, ridx = _tc_stage(x2, rm_padded, pw)
    bs2 = binary_set.reshape(2 ** (BITS - 3) // _ROW, _ROW)
    seen_i32 = _make_sc_gather()(bs2, idx, ridx)
    return seen_i32 > 0


# speedup vs baseline: 1.1814x; 1.1814x over previous
import functools

import jax
import jax.numpy as jnp
from jax import lax
from jax.experimental import pallas as pl
from jax.experimental.pallas import tpu as pltpu
from jax.experimental.pallas import tpu_sc as plsc

BITS = 24
B = 16384
D = 512
LANES = 128
BLK = 2048

_NC = 2
_NS = 16
_L = 16
_NW = _NC * _NS
_PER = B // _NW
_CHUNK = 128
_NSTREAM = _PER // _CHUNK


def _hash_tc(x_ref, rm_ref, pw_ref, idx_ref):
    prod = jnp.dot(x_ref[...], rm_ref[...],
                   preferred_element_type=jnp.float32)
    vals = jnp.where(prod < 0.0, pw_ref[...], 0)
    idx_ref[...] = jnp.sum(vals, axis=1, keepdims=True)


def _tc_stage(x, rm_padded, pw):
    grid = (B // BLK,)
    idx = pl.pallas_call(
        _hash_tc,
        grid=grid,
        in_specs=[
            pl.BlockSpec((BLK, D), lambda i: (i, 0)),
            pl.BlockSpec((D, LANES), lambda i: (0, 0)),
            pl.BlockSpec((1, LANES), lambda i: (0, 0)),
        ],
        out_specs=pl.BlockSpec((BLK, 1), lambda i: (i, 0)),
        out_shape=jax.ShapeDtypeStruct((B, 1), jnp.int32),
    )(x, rm_padded, pw)
    return idx.reshape(B)


_ROW = 128


def _sc_gather_body(bs_hbm, idx_hbm, out_hbm,
                    idx_v, ridx_v, rows_v, out_v, sem):
    table = bs_hbm.bitcast(jnp.int32)
    wid = lax.axis_index("s") * _NC + lax.axis_index("c")
    base = wid * _PER
    lane = lax.iota(jnp.int32, _L)
    for j in range(_NSTREAM):
        pltpu.sync_copy(idx_hbm.at[pl.ds(base + j * _CHUNK, _CHUNK)],
                        idx_v.at[j])
    for j in range(_NSTREAM):
        for t in range(_CHUNK // _L):
            iv = idx_v[j, pl.ds(t * _L, _L)]
            ridx_v[j, pl.ds(t * _L, _L)] = lax.shift_right_logical(iv, 12)
    cps = [pltpu.async_copy(table.at[ridx_v.at[j]], rows_v.at[j], sem)
           for j in range(_NSTREAM)]
    for cp in cps:
        cp.wait()

    def _group(jt, carry):
        j = lax.shift_right_logical(jt, 3)
        t = jt & 7
        iv = idx_v[j, pl.ds(t * _L, _L)]
        cv = lax.shift_right_logical(iv, 3) & (_ROW - 1)
        gv = lax.shift_right_logical(cv, 4)
        pv = cv & (_L - 1)
        acc = jnp.zeros((_L,), jnp.int32)
        for r in range(_L):
            g_r = gv[r]
            sub = jnp.zeros((_L,), jnp.int32)
            for g in range(_ROW // _L):
                v = rows_v[j, t * _L + r, pl.ds(g * _L, _L)]
                sub = jnp.where(g_r == g, v, sub)
            acc = jnp.where(lane == r, sub[pv], acc)
        bitpos = lax.shift_left(lax.shift_right_logical(iv, 10) & 3, 3) \
            | (iv & 7)
        m = lax.shift_left(jnp.full((_L,), 1, jnp.int32), bitpos)
        out_v[j, pl.ds(t * _L, _L)] = jnp.where((acc & m) != 0, 1, 0)
        return carry

    lax.fori_loop(0, _NSTREAM * (_CHUNK // _L), _group, 0)
    for j in range(_NSTREAM):
        pltpu.sync_copy(out_v.at[j],
                        out_hbm.at[pl.ds(base + j * _CHUNK, _CHUNK)])


@functools.lru_cache(maxsize=None)
def _make_sc_gather():
    return pl.kernel(
        _sc_gather_body,
        mesh=plsc.VectorSubcoreMesh(core_axis_name="c", subcore_axis_name="s"),
        out_type=jax.ShapeDtypeStruct((B,), jnp.int32),
        scratch_types=[
            pltpu.VMEM((_NSTREAM, _CHUNK), jnp.int32),
            pltpu.VMEM((_NSTREAM, _CHUNK), jnp.int32),
            pltpu.VMEM((_NSTREAM, _CHUNK, _ROW), jnp.int32),
            pltpu.VMEM((_NSTREAM, _CHUNK), jnp.int32),
            pltpu.SemaphoreType.DMA,
        ],
    )


def kernel(x, random_matrix, binary_set, is_training, test_local_stats):
    x2 = jnp.reshape(x, (B, D))
    rm_padded = jnp.pad(random_matrix, ((0, 0), (0, LANES - BITS)))
    pw = jnp.pad((2 ** jnp.arange(BITS, dtype=jnp.int32))[None, :],
                 ((0, 0), (0, LANES - BITS)))
    idx = _tc_stage(x2, rm_padded, pw)
    bs2 = binary_set.reshape(2 ** (BITS - 3) // _ROW, _ROW)
    seen_i32 = _make_sc_gather()(bs2, idx)
    return seen_i32 > 0


# speedup vs baseline: 1.3683x; 1.1582x over previous
import functools

import jax
import jax.numpy as jnp
from jax import lax
from jax.experimental import pallas as pl
from jax.experimental.pallas import tpu as pltpu
from jax.experimental.pallas import tpu_sc as plsc

BITS = 24
B = 16384
D = 512
LANES = 128
BLK = 2048

_NC = 2
_NS = 16
_L = 16
_NW = _NC * _NS
_PER = B // _NW
_CHUNK = 128
_NSTREAM = _PER // _CHUNK


def _hash_tc(x_ref, rm_ref, pw_ref, idx_ref):
    prod = lax.dot_general(rm_ref[...], x_ref[...],
                           (((0,), (1,)), ((), ())),
                           preferred_element_type=jnp.float32)
    vals = jnp.where(prod < 0.0, pw_ref[...], 0)
    idx_ref[...] = jnp.sum(vals, axis=0, keepdims=True)[None]


def _tc_stage(x, rm_padded, pw_col):
    grid = (B // BLK,)
    idx = pl.pallas_call(
        _hash_tc,
        grid=grid,
        in_specs=[
            pl.BlockSpec((BLK, D), lambda i: (i, 0)),
            pl.BlockSpec((D, LANES), lambda i: (0, 0)),
            pl.BlockSpec((LANES, 1), lambda i: (0, 0)),
        ],
        out_specs=pl.BlockSpec((1, 1, BLK), lambda i: (i, 0, 0)),
        out_shape=jax.ShapeDtypeStruct((B // BLK, 1, BLK), jnp.int32),
    )(x, rm_padded, pw_col)
    return idx.reshape(B)


_ROW = 128


def _sc_gather_body(bs_hbm, idx_hbm, out_hbm,
                    idx_v, ridx_v, rows_v, out_v, sem):
    table = bs_hbm.bitcast(jnp.int32)
    wid = lax.axis_index("s") * _NC + lax.axis_index("c")
    base = wid * _PER
    lane = lax.iota(jnp.int32, _L)
    for j in range(_NSTREAM):
        pltpu.sync_copy(idx_hbm.at[pl.ds(base + j * _CHUNK, _CHUNK)],
                        idx_v.at[j])
    for j in range(_NSTREAM):
        for t in range(_CHUNK // _L):
            iv = idx_v[j, pl.ds(t * _L, _L)]
            ridx_v[j, pl.ds(t * _L, _L)] = lax.shift_right_logical(iv, 12)
    cps = [pltpu.async_copy(table.at[ridx_v.at[j]], rows_v.at[j], sem)
           for j in range(_NSTREAM)]
    for cp in cps:
        cp.wait()

    def _group(jt, carry):
        j = lax.shift_right_logical(jt, 3)
        t = jt & 7
        iv = idx_v[j, pl.ds(t * _L, _L)]
        cv = lax.shift_right_logical(iv, 3) & (_ROW - 1)
        gv = lax.shift_right_logical(cv, 4)
        pv = cv & (_L - 1)
        acc = jnp.zeros((_L,), jnp.int32)
        for r in range(_L):
            g_r = gv[r]
            sub = jnp.zeros((_L,), jnp.int32)
            for g in range(_ROW // _L):
                v = rows_v[j, t * _L + r, pl.ds(g * _L, _L)]
                sub = jnp.where(g_r == g, v, sub)
            acc = jnp.where(lane == r, sub[pv], acc)
        bitpos = lax.shift_left(lax.shift_right_logical(iv, 10) & 3, 3) \
            | (iv & 7)
        m = lax.shift_left(jnp.full((_L,), 1, jnp.int32), bitpos)
        out_v[j, pl.ds(t * _L, _L)] = jnp.where((acc & m) != 0, 1, 0)
        return carry

    lax.fori_loop(0, _NSTREAM * (_CHUNK // _L), _group, 0)
    for j in range(_NSTREAM):
        pltpu.sync_copy(out_v.at[j],
                        out_hbm.at[pl.ds(base + j * _CHUNK, _CHUNK)])


@functools.lru_cache(maxsize=None)
def _make_sc_gather():
    return pl.kernel(
        _sc_gather_body,
        mesh=plsc.VectorSubcoreMesh(core_axis_name="c", subcore_axis_name="s"),
        out_type=jax.ShapeDtypeStruct((B,), jnp.int32),
        scratch_types=[
            pltpu.VMEM((_NSTREAM, _CHUNK), jnp.int32),
            pltpu.VMEM((_NSTREAM, _CHUNK), jnp.int32),
            pltpu.VMEM((_NSTREAM, _CHUNK, _ROW), jnp.int32),
            pltpu.VMEM((_NSTREAM, _CHUNK), jnp.int32),
            pltpu.SemaphoreType.DMA,
        ],
    )


def kernel(x, random_matrix, binary_set, is_training, test_local_stats):
    x2 = jnp.reshape(x, (B, D))
    rm_padded = jnp.pad(random_matrix, ((0, 0), (0, LANES - BITS)))
    pw_col = jnp.pad((2 ** jnp.arange(BITS, dtype=jnp.int32))[:, None],
                     ((0, LANES - BITS), (0, 0)))
    idx = _tc_stage(x2, rm_padded, pw_col)
    bs2 = binary_set.reshape(2 ** (BITS - 3) // _ROW, _ROW)
    seen_i32 = _make_sc_gather()(bs2, idx)
    return seen_i32 > 0


# speedup vs baseline: 1.4166x; 1.0353x over previous
import functools

import jax
import jax.numpy as jnp
from jax import lax
from jax.experimental import pallas as pl
from jax.experimental.pallas import tpu as pltpu
from jax.experimental.pallas import tpu_sc as plsc

BITS = 24
B = 16384
D = 512
LANES = 128
BLK = 4096

_NC = 2
_NS = 16
_L = 16
_NW = _NC * _NS
_PER = B // _NW
_CHUNK = 128
_NSTREAM = _PER // _CHUNK


def _hash_tc(x_ref, rm_ref, pw_ref, idx_ref):
    prod = lax.dot_general(rm_ref[...], x_ref[...],
                           (((0,), (1,)), ((), ())),
                           preferred_element_type=jnp.float32)
    vals = jnp.where(prod < 0.0, pw_ref[...], 0)
    idx_ref[...] = jnp.sum(vals, axis=0, keepdims=True)[None]


def _tc_stage(x, rm_padded, pw_col):
    grid = (B // BLK,)
    idx = pl.pallas_call(
        _hash_tc,
        grid=grid,
        in_specs=[
            pl.BlockSpec((BLK, D), lambda i: (i, 0)),
            pl.BlockSpec((D, LANES), lambda i: (0, 0)),
            pl.BlockSpec((LANES, 1), lambda i: (0, 0)),
        ],
        out_specs=pl.BlockSpec((1, 1, BLK), lambda i: (i, 0, 0)),
        out_shape=jax.ShapeDtypeStruct((B // BLK, 1, BLK), jnp.int32),
    )(x, rm_padded, pw_col)
    return idx.reshape(B)


_ROW = 128


def _sc_gather_body(bs_hbm, idx_hbm, out_hbm,
                    idx_v, ridx_v, rows_v, out_v, sem):
    table = bs_hbm.bitcast(jnp.int32)
    wid = lax.axis_index("s") * _NC + lax.axis_index("c")
    base = wid * _PER
    lane = lax.iota(jnp.int32, _L)
    for j in range(_NSTREAM):
        pltpu.sync_copy(idx_hbm.at[pl.ds(base + j * _CHUNK, _CHUNK)],
                        idx_v.at[j])
    for j in range(_NSTREAM):
        for t in range(_CHUNK // _L):
            iv = idx_v[j, pl.ds(t * _L, _L)]
            ridx_v[j, pl.ds(t * _L, _L)] = lax.shift_right_logical(iv, 12)
    cps = [pltpu.async_copy(table.at[ridx_v.at[j]], rows_v.at[j], sem)
           for j in range(_NSTREAM)]
    for cp in cps:
        cp.wait()

    def _group(jt, carry):
        j = lax.shift_right_logical(jt, 3)
        t = jt & 7
        iv = idx_v[j, pl.ds(t * _L, _L)]
        cv = lax.shift_right_logical(iv, 3) & (_ROW - 1)
        gv = lax.shift_right_logical(cv, 4)
        pv = cv & (_L - 1)
        acc = jnp.zeros((_L,), jnp.int32)
        for r in range(_L):
            g_r = gv[r]
            sub = jnp.zeros((_L,), jnp.int32)
            for g in range(_ROW // _L):
                v = rows_v[j, t * _L + r, pl.ds(g * _L, _L)]
                sub = jnp.where(g_r == g, v, sub)
            acc = jnp.where(lane == r, sub[pv], acc)
        bitpos = lax.shift_left(lax.shift_right_logical(iv, 10) & 3, 3) \
            | (iv & 7)
        m = lax.shift_left(jnp.full((_L,), 1, jnp.int32), bitpos)
        out_v[j, pl.ds(t * _L, _L)] = jnp.where((acc & m) != 0, 1, 0)
        return carry

    lax.fori_loop(0, _NSTREAM * (_CHUNK // _L), _group, 0)
    for j in range(_NSTREAM):
        pltpu.sync_copy(out_v.at[j],
                        out_hbm.at[pl.ds(base + j * _CHUNK, _CHUNK)])


@functools.lru_cache(maxsize=None)
def _make_sc_gather():
    return pl.kernel(
        _sc_gather_body,
        mesh=plsc.VectorSubcoreMesh(core_axis_name="c", subcore_axis_name="s"),
        out_type=jax.ShapeDtypeStruct((B,), jnp.int32),
        scratch_types=[
            pltpu.VMEM((_NSTREAM, _CHUNK), jnp.int32),
            pltpu.VMEM((_NSTREAM, _CHUNK), jnp.int32),
            pltpu.VMEM((_NSTREAM, _CHUNK, _ROW), jnp.int32),
            pltpu.VMEM((_NSTREAM, _CHUNK), jnp.int32),
            pltpu.SemaphoreType.DMA,
        ],
    )


def kernel(x, random_matrix, binary_set, is_training, test_local_stats):
    x2 = jnp.reshape(x, (B, D))
    rm_padded = jnp.pad(random_matrix, ((0, 0), (0, LANES - BITS)))
    pw_col = jnp.pad((2 ** jnp.arange(BITS, dtype=jnp.int32))[:, None],
                     ((0, LANES - BITS), (0, 0)))
    idx = _tc_stage(x2, rm_padded, pw_col)
    bs2 = binary_set.reshape(2 ** (BITS - 3) // _ROW, _ROW)
    seen_i32 = _make_sc_gather()(bs2, idx)
    return seen_i32 > 0
